# baseline (speedup 1.0000x reference)
import dataclasses
import functools
import math

import jax
import jax.numpy as jnp
from jax import lax
from jax.experimental import pallas as pl
from jax.experimental.pallas import tpu as pltpu
from jax.experimental.pallas import tpu_sc as plsc

XPAD = 4


def _pack_bf16_pair(lo_f32, hi_f32):
    lo = lax.convert_element_type(
        lax.bitcast_convert_type(lo_f32.astype(jnp.bfloat16), jnp.uint16), jnp.uint32)
    hi = lax.convert_element_type(
        lax.bitcast_convert_type(hi_f32.astype(jnp.bfloat16), jnp.uint16), jnp.uint32)
    return lax.bitcast_convert_type((hi << 16) | lo, jnp.int32)


def _unpack_bf16_pair(packed_i32):
    u = lax.bitcast_convert_type(packed_i32, jnp.uint32)
    lo = lax.bitcast_convert_type(
        lax.convert_element_type(u & jnp.uint32(0xFFFF), jnp.uint16), jnp.bfloat16)
    hi = lax.bitcast_convert_type(
        lax.convert_element_type(u >> 16, jnp.uint16), jnp.bfloat16)
    return lo.astype(jnp.float32), hi.astype(jnp.float32)


def _build_table(x2, idx2, Wk, bk, Wv, bv, N, BT):
    BNT, C = x2.shape
    K = idx2.shape[1]

    def body(x_ref, idx_ref, wk_ref, bk_ref, wv_ref, bv_ref, tab_ref, idxa_ref):
        xb = x_ref[...].astype(jnp.bfloat16)
        kf = jnp.dot(xb, wk_ref[...].astype(jnp.bfloat16),
                     preferred_element_type=jnp.float32) + bk_ref[...]
        vf = jnp.dot(xb, wv_ref[...].astype(jnp.bfloat16),
                     preferred_element_type=jnp.float32) + bv_ref[...]
        tab_ref[...] = _pack_bf16_pair(kf, vf)
        off = (pl.program_id(0) * BT // N) * N
        idxa_ref[...] = idx_ref[...] + off

    grid = (BNT // BT,)
    return pl.pallas_call(
        body,
        grid=grid,
        in_specs=[
            pl.BlockSpec((BT, C), lambda i: (i, 0)),
            pl.BlockSpec((BT, K), lambda i: (i, 0)),
            pl.BlockSpec((C, C), lambda i: (0, 0)),
            pl.BlockSpec((1, C), lambda i: (0, 0)),
            pl.BlockSpec((C, C), lambda i: (0, 0)),
            pl.BlockSpec((1, C), lambda i: (0, 0)),
        ],
        out_specs=[
            pl.BlockSpec((BT, C), lambda i: (i, 0)),
            pl.BlockSpec((BT, K), lambda i: (i, 0)),
        ],
        out_shape=[
            jax.ShapeDtypeStruct((BNT, C), jnp.int32),
            jax.ShapeDtypeStruct((BNT, K), jnp.int32),
        ],
    )(x2, idx2, Wk, bk, Wv, bv)


def _sc_gather(table, idx_flat, xyzT, N, K):
    ROWS = idx_flat.shape[0]
    C = table.shape[1]
    NW = 32
    CH = 128
    L = 16
    RPW = ROWS // NW
    NCH = RPW // CH
    NKB = N * K
    mesh = plsc.VectorSubcoreMesh(core_axis_name="c", subcore_axis_name="s")
    cp = pltpu.CompilerParams()
    if "needs_layout_passes" in pltpu.CompilerParams.__dataclass_fields__:
        cp = dataclasses.replace(cp, needs_layout_passes=False)

    B = ROWS // NKB

    @functools.partial(
        pl.kernel,
        compiler_params=cp,
        out_type=[
            jax.ShapeDtypeStruct((B, NKB, C), jnp.int32),
            jax.ShapeDtypeStruct((ROWS * XPAD,), jnp.float32),
        ],
        mesh=mesh,
        scratch_types=[
            pltpu.VMEM((N,), jnp.float32),
            pltpu.VMEM((N,), jnp.float32),
            pltpu.VMEM((N,), jnp.float32),
            pltpu.VMEM((CH,), jnp.int32),
            pltpu.VMEM((CH,), jnp.int32),
            pltpu.VMEM((CH, C), jnp.int32),
            pltpu.VMEM((CH, C), jnp.int32),
            pltpu.VMEM((CH * XPAD,), jnp.float32),
            pltpu.VMEM((CH * XPAD,), jnp.float32),
            pltpu.SemaphoreType.DMA,
            pltpu.SemaphoreType.DMA,
            pltpu.SemaphoreType.DMA,
            pltpu.SemaphoreType.DMA,
            pltpu.SemaphoreType.DMA,
            pltpu.SemaphoreType.DMA,
        ],
    )
    def gather_k(tab_hbm, idx_hbm, xyzT_hbm, out_hbm, nx_hbm,
                 cx_v, cy_v, cz_v, idx0, idx1, rows0, rows1, nx0, nx1,
                 is0, is1, gs0, gs1, ws0, ws1):
        idx_v = (idx0, idx1)
        rows_v = (rows0, rows1)
        nx_v = (nx0, nx1)
        isem = (is0, is1)
        gsem = (gs0, gs1)
        wsem = (ws0, ws1)
        wid = lax.axis_index("s") * 2 + lax.axis_index("c")
        base = wid * RPW
        batch = base // NKB
        boff = batch * N

        pltpu.sync_copy(xyzT_hbm.at[pl.ds((batch * 3 + 0) * N, N)], cx_v)
        pltpu.sync_copy(xyzT_hbm.at[pl.ds((batch * 3 + 1) * N, N)], cy_v)
        pltpu.sync_copy(xyzT_hbm.at[pl.ds((batch * 3 + 2) * N, N)], cz_v)

        zeros = jnp.zeros((L,), jnp.float32)
        for s0 in range(2):
            @pl.loop(0, CH * XPAD // L)
            def _(z, s0=s0):
                nx_v[s0][pl.ds(z * L, L)] = zeros

        def idx_src(i):
            return idx_hbm.at[pl.ds(base + i * CH, CH)]

        def rows_dst(i):
            return out_hbm.at[batch, pl.ds(base + i * CH - batch * NKB, CH)]

        def nx_dst(i):
            return nx_hbm.at[pl.ds((base + i * CH) * XPAD, CH * XPAD)]

        def compute_nx(s):
            for j in range(CH // L):
                nb = idx_v[s][pl.ds(j * L, L)] - boff
                flat = (lax.iota(jnp.int32, L) + (j * L)) * XPAD
                gx = plsc.load_gather(cx_v, [nb])
                gy = plsc.load_gather(cy_v, [nb])
                gz = plsc.load_gather(cz_v, [nb])
                plsc.store_scatter(nx_v[s], [flat], gx)
                plsc.store_scatter(nx_v[s], [flat + 1], gy)
                plsc.store_scatter(nx_v[s], [flat + 2], gz)

        pltpu.async_copy(idx_src(0), idx_v[0], isem[0])

        @pl.loop(0, NCH, step=2)
        def _(i0):
            for par in range(2):
                s, o = par, 1 - par
                i = i0 + par
                @pl.when(i >= 1)
                def _():
                    pltpu.make_async_copy(
                        tab_hbm.at[idx_v[o]], rows_v[o], gsem[o]).wait()
                    pltpu.async_copy(rows_v[o], rows_dst(i - 1), wsem[o])
                    pltpu.async_copy(nx_v[o], nx_dst(i - 1), wsem[o])
                @pl.when(i + 1 < NCH)
                def _():
                    pltpu.async_copy(idx_src(i + 1), idx_v[o], isem[o])
                @pl.when(i >= 2)
                def _():
                    pltpu.make_async_copy(rows_v[s], rows_dst(i - 2), wsem[s]).wait()
                    pltpu.make_async_copy(nx_v[s], nx_dst(i - 2), wsem[s]).wait()
                pltpu.make_async_copy(idx_src(i), idx_v[s], isem[s]).wait()
                compute_nx(s)
                pltpu.async_copy(tab_hbm.at[idx_v[s]], rows_v[s], gsem[s])

        last = NCH - 1
        sl = last % 2
        pltpu.make_async_copy(tab_hbm.at[idx_v[sl]], rows_v[sl], gsem[sl]).wait()
        pltpu.async_copy(rows_v[sl], rows_dst(last), wsem[sl])
        pltpu.async_copy(nx_v[sl], nx_dst(last), wsem[sl])
        pltpu.make_async_copy(rows_v[1 - sl], rows_dst(last - 1), wsem[1 - sl]).wait()
        pltpu.make_async_copy(nx_v[1 - sl], nx_dst(last - 1), wsem[1 - sl]).wait()
        pltpu.make_async_copy(rows_v[sl], rows_dst(last), wsem[sl]).wait()
        pltpu.make_async_copy(nx_v[sl], nx_dst(last), wsem[sl]).wait()

    return gather_k(table, idx_flat, xyzT)


def _attention(x, xyzp3, gath3, nx3, Wq, bq, Wm1p, bm1, Wm2r, bm2r, Wo, bo,
               hd, BP):
    B, N, C = x.shape
    K = gath3.shape[1] // N
    scale = 1.0 / math.sqrt(hd)

    def body(x_ref, xyz_ref, g_ref, nx_ref, wq_ref, bq_ref, wm1_ref, bm1_ref,
             wm2_ref, bm2_ref, wo_ref, bo_ref, o_ref):
        bf = jnp.bfloat16
        xb = x_ref[0]
        q = jnp.dot(xb, wq_ref[...], preferred_element_type=jnp.float32) + bq_ref[...]
        q_bf = q.astype(bf)
        u = lax.bitcast_convert_type(g_ref[0], jnp.uint32)
        kn_bf = lax.bitcast_convert_type(
            lax.convert_element_type(u & jnp.uint32(0xFFFF), jnp.uint16), bf)
        vn_bf = lax.bitcast_convert_type(
            lax.convert_element_type(u >> 16, jnp.uint16), bf)

        nx = nx_ref[0]
        rel = xyz_ref[0][:, None, :] - nx.reshape(BP, K, XPAD)
        rel_bf = rel.reshape(BP * K, XPAD).astype(bf)
        h1 = jnp.dot(rel_bf, wm1_ref[...].astype(bf),
                     preferred_element_type=jnp.float32) + bm1_ref[...]
        h1_bf = jnp.maximum(h1, 0.0).astype(bf)
        hb = jnp.dot(h1_bf, wm2_ref[...].astype(bf),
                     preferred_element_type=jnp.float32) + bm2_ref[...]

        ce = lax.broadcasted_iota(jnp.int32, (C, C), 0) // hd
        je = lax.broadcasted_iota(jnp.int32, (C, C), 1) // hd
        E2 = jnp.where(ce == je, scale, 0.0).astype(bf)
        prod = (kn_bf.reshape(BP, K, C) * q_bf[:, None, :]).reshape(BP * K, C)
        s = jnp.dot(prod, E2, preferred_element_type=jnp.float32) + hb

        s3 = s.reshape(BP, K, C)
        m = jnp.max(s3, axis=1, keepdims=True)
        e = jnp.exp(s3 - m)
        den = jnp.sum(e, axis=1, keepdims=True)
        attn_bf = (e / den).astype(bf)

        oa = (attn_bf * vn_bf.reshape(BP, K, C)).sum(axis=1).astype(jnp.float32)
        o_ref[0] = jnp.dot(oa, wo_ref[...], preferred_element_type=jnp.float32) + bo_ref[...]

    grid = (B, N // BP)
    return pl.pallas_call(
        body,
        grid=grid,
        in_specs=[
            pl.BlockSpec((1, BP, C), lambda b, i: (b, i, 0)),
            pl.BlockSpec((1, BP, XPAD), lambda b, i: (b, i, 0)),
            pl.BlockSpec((1, BP * K, C), lambda b, i: (b, i, 0)),
            pl.BlockSpec((1, BP * K, XPAD), lambda b, i: (b, i, 0)),
            pl.BlockSpec((C, C), lambda b, i: (0, 0)),
            pl.BlockSpec((1, C), lambda b, i: (0, 0)),
            pl.BlockSpec((XPAD, 32), lambda b, i: (0, 0)),
            pl.BlockSpec((1, 32), lambda b, i: (0, 0)),
            pl.BlockSpec((32, C), lambda b, i: (0, 0)),
            pl.BlockSpec((1, C), lambda b, i: (0, 0)),
            pl.BlockSpec((C, C), lambda b, i: (0, 0)),
            pl.BlockSpec((1, C), lambda b, i: (0, 0)),
        ],
        out_specs=pl.BlockSpec((1, BP, C), lambda b, i: (b, i, 0)),
        out_shape=jax.ShapeDtypeStruct((B, N, C), jnp.float32),
    )(x, xyzp3, gath3, nx3, Wq, bq, Wm1p, bm1, Wm2r, bm2r, Wo, bo)


def kernel(x, xyz, idx, Wq, bq, Wk, bk, Wv, bv, Wo, bo, Wm1, bm1, Wm2, bm2):
    B, N, C = x.shape
    K = idx.shape[2]
    H = Wm2.shape[1]

    x2 = x.reshape(B * N, C)
    idx2 = idx.reshape(B * N, K).astype(jnp.int32)
    xyzT = jnp.transpose(xyz, (0, 2, 1)).reshape(B * 3 * N)
    xyzp = jnp.pad(xyz, ((0, 0), (0, 0), (0, XPAD - 3)))
    Wm1p = jnp.pad(Wm1, ((0, XPAD - 3), (0, 0)))

    hd = C // H
    Wm2r = jnp.repeat(Wm2, hd, axis=1)
    bm2r = jnp.repeat(bm2.reshape(1, H), hd, axis=1)

    table, idxa = _build_table(x2, idx2, Wk, bk.reshape(1, C),
                               Wv, bv.reshape(1, C), N, BT=1024)
    gath3, nx = _sc_gather(table, idxa.reshape(B * N * K), xyzT, N, K)
    nx3 = nx.reshape(B, N * K, XPAD)

    out = _attention(x, xyzp, gath3, nx3,
                     Wq, bq.reshape(1, C), Wm1p, bm1.reshape(1, 32),
                     Wm2r, bm2r, Wo, bo.reshape(1, C), hd=hd, BP=256)
    return out


# speedup vs baseline: 1.3146x; 1.3146x over previous
import dataclasses
import functools
import math

import jax
import jax.numpy as jnp
from jax import lax
from jax.experimental import pallas as pl
from jax.experimental.pallas import tpu as pltpu
from jax.experimental.pallas import tpu_sc as plsc

XPAD = 4


def _pack_bf16_pair(lo_f32, hi_f32):
    lo = lax.convert_element_type(
        lax.bitcast_convert_type(lo_f32.astype(jnp.bfloat16), jnp.uint16), jnp.uint32)
    hi = lax.convert_element_type(
        lax.bitcast_convert_type(hi_f32.astype(jnp.bfloat16), jnp.uint16), jnp.uint32)
    return lax.bitcast_convert_type((hi << 16) | lo, jnp.int32)


def _unpack_bf16_pair(packed_i32):
    u = lax.bitcast_convert_type(packed_i32, jnp.uint32)
    lo = lax.bitcast_convert_type(
        lax.convert_element_type(u & jnp.uint32(0xFFFF), jnp.uint16), jnp.bfloat16)
    hi = lax.bitcast_convert_type(
        lax.convert_element_type(u >> 16, jnp.uint16), jnp.bfloat16)
    return lo.astype(jnp.float32), hi.astype(jnp.float32)


def _build_table(x2, idx2, Wk, bk, Wv, bv, N, BT):
    BNT, C = x2.shape
    K = idx2.shape[1]

    def body(x_ref, idx_ref, wk_ref, bk_ref, wv_ref, bv_ref, tab_ref, idxa_ref):
        xb = x_ref[...].astype(jnp.bfloat16)
        kf = jnp.dot(xb, wk_ref[...].astype(jnp.bfloat16),
                     preferred_element_type=jnp.float32) + bk_ref[...]
        vf = jnp.dot(xb, wv_ref[...].astype(jnp.bfloat16),
                     preferred_element_type=jnp.float32) + bv_ref[...]
        tab_ref[...] = _pack_bf16_pair(kf, vf)
        off = (pl.program_id(0) * BT // N) * N
        idxa_ref[...] = idx_ref[...] + off

    grid = (BNT // BT,)
    return pl.pallas_call(
        body,
        grid=grid,
        in_specs=[
            pl.BlockSpec((BT, C), lambda i: (i, 0)),
            pl.BlockSpec((BT, K), lambda i: (i, 0)),
            pl.BlockSpec((C, C), lambda i: (0, 0)),
            pl.BlockSpec((1, C), lambda i: (0, 0)),
            pl.BlockSpec((C, C), lambda i: (0, 0)),
            pl.BlockSpec((1, C), lambda i: (0, 0)),
        ],
        out_specs=[
            pl.BlockSpec((BT, C), lambda i: (i, 0)),
            pl.BlockSpec((BT, K), lambda i: (i, 0)),
        ],
        out_shape=[
            jax.ShapeDtypeStruct((BNT, C), jnp.int32),
            jax.ShapeDtypeStruct((BNT, K), jnp.int32),
        ],
    )(x2, idx2, Wk, bk, Wv, bv)


def _sc_gather(table, idx_flat, xyzT, N, K):
    ROWS = idx_flat.shape[0]
    C = table.shape[1]
    NW = 32
    CH = 128
    L = 16
    RPW = ROWS // NW
    NCH = RPW // CH
    NKB = N * K
    mesh = plsc.VectorSubcoreMesh(core_axis_name="c", subcore_axis_name="s")
    cp = pltpu.CompilerParams()
    if "needs_layout_passes" in pltpu.CompilerParams.__dataclass_fields__:
        cp = dataclasses.replace(cp, needs_layout_passes=False)

    B = ROWS // NKB

    @functools.partial(
        pl.kernel,
        compiler_params=cp,
        out_type=[
            jax.ShapeDtypeStruct((B, NKB, C), jnp.int32),
            jax.ShapeDtypeStruct((B, NKB, XPAD), jnp.float32),
        ],
        mesh=mesh,
        scratch_types=[
            pltpu.VMEM((N,), jnp.float32),
            pltpu.VMEM((N,), jnp.float32),
            pltpu.VMEM((N,), jnp.float32),
            pltpu.VMEM((CH,), jnp.int32),
            pltpu.VMEM((CH,), jnp.int32),
            pltpu.VMEM((CH, C), jnp.int32),
            pltpu.VMEM((CH, C), jnp.int32),
            pltpu.VMEM((CH, XPAD), jnp.float32),
            pltpu.VMEM((CH, XPAD), jnp.float32),
            pltpu.SemaphoreType.DMA,
            pltpu.SemaphoreType.DMA,
            pltpu.SemaphoreType.DMA,
            pltpu.SemaphoreType.DMA,
            pltpu.SemaphoreType.DMA,
            pltpu.SemaphoreType.DMA,
        ],
    )
    def gather_k(tab_hbm, idx_hbm, xyzT_hbm, out_hbm, nx_hbm,
                 cx_v, cy_v, cz_v, idx0, idx1, rows0, rows1, nx0, nx1,
                 is0, is1, gs0, gs1, ws0, ws1):
        idx_v = (idx0, idx1)
        rows_v = (rows0, rows1)
        nx_v = (nx0, nx1)
        isem = (is0, is1)
        gsem = (gs0, gs1)
        wsem = (ws0, ws1)
        wid = lax.axis_index("s") * 2 + lax.axis_index("c")
        base = wid * RPW
        batch = base // NKB
        boff = batch * N

        pltpu.sync_copy(xyzT_hbm.at[pl.ds((batch * 3 + 0) * N, N)], cx_v)
        pltpu.sync_copy(xyzT_hbm.at[pl.ds((batch * 3 + 1) * N, N)], cy_v)
        pltpu.sync_copy(xyzT_hbm.at[pl.ds((batch * 3 + 2) * N, N)], cz_v)

        zeros = jnp.zeros((L,), jnp.float32)

        def idx_src(i):
            return idx_hbm.at[pl.ds(base + i * CH, CH)]

        def rows_dst(i):
            return out_hbm.at[batch, pl.ds(base + i * CH - batch * NKB, CH)]

        def nx_dst(i):
            return nx_hbm.at[batch, pl.ds(base + i * CH - batch * NKB, CH)]

        def compute_nx(s):
            for j in range(CH // L):
                nb = idx_v[s][pl.ds(j * L, L)] - boff
                rows16 = lax.iota(jnp.int32, L) + (j * L)
                gx = plsc.load_gather(cx_v, [nb])
                gy = plsc.load_gather(cy_v, [nb])
                gz = plsc.load_gather(cz_v, [nb])
                plsc.store_scatter(nx_v[s], [rows16, jnp.full((L,), 0, jnp.int32)], gx)
                plsc.store_scatter(nx_v[s], [rows16, jnp.full((L,), 1, jnp.int32)], gy)
                plsc.store_scatter(nx_v[s], [rows16, jnp.full((L,), 2, jnp.int32)], gz)
                plsc.store_scatter(nx_v[s], [rows16, jnp.full((L,), 3, jnp.int32)], zeros)

        pltpu.async_copy(idx_src(0), idx_v[0], isem[0])

        @pl.loop(0, NCH, step=2)
        def _(i0):
            for par in range(2):
                s, o = par, 1 - par
                i = i0 + par
                @pl.when(i >= 1)
                def _():
                    pltpu.make_async_copy(
                        tab_hbm.at[idx_v[o]], rows_v[o], gsem[o]).wait()
                    pltpu.async_copy(rows_v[o], rows_dst(i - 1), wsem[o])
                    pltpu.async_copy(nx_v[o], nx_dst(i - 1), wsem[o])
                @pl.when(i + 1 < NCH)
                def _():
                    pltpu.async_copy(idx_src(i + 1), idx_v[o], isem[o])
                @pl.when(i >= 2)
                def _():
                    pltpu.make_async_copy(rows_v[s], rows_dst(i - 2), wsem[s]).wait()
                    pltpu.make_async_copy(nx_v[s], nx_dst(i - 2), wsem[s]).wait()
                pltpu.make_async_copy(idx_src(i), idx_v[s], isem[s]).wait()
                compute_nx(s)
                pltpu.async_copy(tab_hbm.at[idx_v[s]], rows_v[s], gsem[s])

        last = NCH - 1
        sl = last % 2
        pltpu.make_async_copy(tab_hbm.at[idx_v[sl]], rows_v[sl], gsem[sl]).wait()
        pltpu.async_copy(rows_v[sl], rows_dst(last), wsem[sl])
        pltpu.async_copy(nx_v[sl], nx_dst(last), wsem[sl])
        pltpu.make_async_copy(rows_v[1 - sl], rows_dst(last - 1), wsem[1 - sl]).wait()
        pltpu.make_async_copy(nx_v[1 - sl], nx_dst(last - 1), wsem[1 - sl]).wait()
        pltpu.make_async_copy(rows_v[sl], rows_dst(last), wsem[sl]).wait()
        pltpu.make_async_copy(nx_v[sl], nx_dst(last), wsem[sl]).wait()

    return gather_k(table, idx_flat, xyzT)


def _attention(x, xyzp3, gath3, nx3, Wq, bq, Wm1p, bm1, Wm2r, bm2r, Wo, bo,
               hd, BP):
    B, N, C = x.shape
    K = gath3.shape[1] // N
    scale = 1.0 / math.sqrt(hd)

    def body(x_ref, xyz_ref, g_ref, nx_ref, wq_ref, bq_ref, wm1_ref, bm1_ref,
             wm2_ref, bm2_ref, wo_ref, bo_ref, o_ref):
        bf = jnp.bfloat16
        xb = x_ref[0]
        q = jnp.dot(xb, wq_ref[...], preferred_element_type=jnp.float32) + bq_ref[...]
        q_bf = q.astype(bf)
        u = lax.bitcast_convert_type(g_ref[0], jnp.uint32)
        kn_bf = lax.bitcast_convert_type(
            lax.convert_element_type(u & jnp.uint32(0xFFFF), jnp.uint16), bf)
        vn_bf = lax.bitcast_convert_type(
            lax.convert_element_type(u >> 16, jnp.uint16), bf)

        nx = nx_ref[0]
        rel = xyz_ref[0][:, None, :] - nx.reshape(BP, K, XPAD)
        rel_bf = rel.reshape(BP * K, XPAD).astype(bf)
        h1 = jnp.dot(rel_bf, wm1_ref[...].astype(bf),
                     preferred_element_type=jnp.float32) + bm1_ref[...]
        h1_bf = jnp.maximum(h1, 0.0).astype(bf)
        hb = jnp.dot(h1_bf, wm2_ref[...].astype(bf),
                     preferred_element_type=jnp.float32) + bm2_ref[...]

        ce = lax.broadcasted_iota(jnp.int32, (C, C), 0) // hd
        je = lax.broadcasted_iota(jnp.int32, (C, C), 1) // hd
        E2 = jnp.where(ce == je, scale, 0.0).astype(bf)
        prod = (kn_bf.reshape(BP, K, C) * q_bf[:, None, :]).reshape(BP * K, C)
        s = jnp.dot(prod, E2, preferred_element_type=jnp.float32) + hb

        s3 = s.reshape(BP, K, C)
        m = jnp.max(s3, axis=1, keepdims=True)
        e = jnp.exp(s3 - m)
        den = jnp.sum(e, axis=1, keepdims=True)
        attn_bf = (e / den).astype(bf)

        oa = (attn_bf * vn_bf.reshape(BP, K, C)).sum(axis=1).astype(jnp.float32)
        o_ref[0] = jnp.dot(oa, wo_ref[...], preferred_element_type=jnp.float32) + bo_ref[...]

    grid = (B, N // BP)
    return pl.pallas_call(
        body,
        grid=grid,
        in_specs=[
            pl.BlockSpec((1, BP, C), lambda b, i: (b, i, 0)),
            pl.BlockSpec((1, BP, XPAD), lambda b, i: (b, i, 0)),
            pl.BlockSpec((1, BP * K, C), lambda b, i: (b, i, 0)),
            pl.BlockSpec((1, BP * K, XPAD), lambda b, i: (b, i, 0)),
            pl.BlockSpec((C, C), lambda b, i: (0, 0)),
            pl.BlockSpec((1, C), lambda b, i: (0, 0)),
            pl.BlockSpec((XPAD, 32), lambda b, i: (0, 0)),
            pl.BlockSpec((1, 32), lambda b, i: (0, 0)),
            pl.BlockSpec((32, C), lambda b, i: (0, 0)),
            pl.BlockSpec((1, C), lambda b, i: (0, 0)),
            pl.BlockSpec((C, C), lambda b, i: (0, 0)),
            pl.BlockSpec((1, C), lambda b, i: (0, 0)),
        ],
        out_specs=pl.BlockSpec((1, BP, C), lambda b, i: (b, i, 0)),
        out_shape=jax.ShapeDtypeStruct((B, N, C), jnp.float32),
    )(x, xyzp3, gath3, nx3, Wq, bq, Wm1p, bm1, Wm2r, bm2r, Wo, bo)


def kernel(x, xyz, idx, Wq, bq, Wk, bk, Wv, bv, Wo, bo, Wm1, bm1, Wm2, bm2):
    B, N, C = x.shape
    K = idx.shape[2]
    H = Wm2.shape[1]

    x2 = x.reshape(B * N, C)
    idx2 = idx.reshape(B * N, K).astype(jnp.int32)
    xyzT = jnp.transpose(xyz, (0, 2, 1)).reshape(B * 3 * N)
    xyzp = jnp.pad(xyz, ((0, 0), (0, 0), (0, XPAD - 3)))
    Wm1p = jnp.pad(Wm1, ((0, XPAD - 3), (0, 0)))

    hd = C // H
    Wm2r = jnp.repeat(Wm2, hd, axis=1)
    bm2r = jnp.repeat(bm2.reshape(1, H), hd, axis=1)

    table, idxa = _build_table(x2, idx2, Wk, bk.reshape(1, C),
                               Wv, bv.reshape(1, C), N, BT=1024)
    gath3, nx3 = _sc_gather(table, idxa.reshape(B * N * K), xyzT, N, K)

    out = _attention(x, xyzp, gath3, nx3,
                     Wq, bq.reshape(1, C), Wm1p, bm1.reshape(1, 32),
                     Wm2r, bm2r, Wo, bo.reshape(1, C), hd=hd, BP=256)
    return out


# speedup vs baseline: 1.3529x; 1.0291x over previous
import dataclasses
import functools
import math

import jax
import jax.numpy as jnp
from jax import lax
from jax.experimental import pallas as pl
from jax.experimental.pallas import tpu as pltpu
from jax.experimental.pallas import tpu_sc as plsc

XPAD = 4


def _pack_bf16_pair(lo_f32, hi_f32):
    lo = lax.convert_element_type(
        lax.bitcast_convert_type(lo_f32.astype(jnp.bfloat16), jnp.uint16), jnp.uint32)
    hi = lax.convert_element_type(
        lax.bitcast_convert_type(hi_f32.astype(jnp.bfloat16), jnp.uint16), jnp.uint32)
    return lax.bitcast_convert_type((hi << 16) | lo, jnp.int32)


def _unpack_bf16_pair(packed_i32):
    u = lax.bitcast_convert_type(packed_i32, jnp.uint32)
    lo = lax.bitcast_convert_type(
        lax.convert_element_type(u & jnp.uint32(0xFFFF), jnp.uint16), jnp.bfloat16)
    hi = lax.bitcast_convert_type(
        lax.convert_element_type(u >> 16, jnp.uint16), jnp.bfloat16)
    return lo.astype(jnp.float32), hi.astype(jnp.float32)


def _build_table(x2, idx2, Wk, bk, Wv, bv, N, BT):
    BNT, C = x2.shape
    K = idx2.shape[1]

    def body(x_ref, idx_ref, wk_ref, bk_ref, wv_ref, bv_ref, tab_ref, idxa_ref):
        xb = x_ref[...].astype(jnp.bfloat16)
        kf = jnp.dot(xb, wk_ref[...].astype(jnp.bfloat16),
                     preferred_element_type=jnp.float32) + bk_ref[...]
        vf = jnp.dot(xb, wv_ref[...].astype(jnp.bfloat16),
                     preferred_element_type=jnp.float32) + bv_ref[...]
        tab_ref[...] = _pack_bf16_pair(kf, vf)
        off = (pl.program_id(0) * BT // N) * N
        idxa_ref[...] = idx_ref[...] + off

    grid = (BNT // BT,)
    return pl.pallas_call(
        body,
        grid=grid,
        in_specs=[
            pl.BlockSpec((BT, C), lambda i: (i, 0)),
            pl.BlockSpec((BT, K), lambda i: (i, 0)),
            pl.BlockSpec((C, C), lambda i: (0, 0)),
            pl.BlockSpec((1, C), lambda i: (0, 0)),
            pl.BlockSpec((C, C), lambda i: (0, 0)),
            pl.BlockSpec((1, C), lambda i: (0, 0)),
        ],
        out_specs=[
            pl.BlockSpec((BT, C), lambda i: (i, 0)),
            pl.BlockSpec((BT, K), lambda i: (i, 0)),
        ],
        out_shape=[
            jax.ShapeDtypeStruct((BNT, C), jnp.int32),
            jax.ShapeDtypeStruct((BNT, K), jnp.int32),
        ],
    )(x2, idx2, Wk, bk, Wv, bv)


def _sc_gather(table, idx_flat, xyzT, N, K, batch):
    ROWS = idx_flat.shape[0]
    C = table.shape[1]
    NW = 32
    CH = 128
    L = 16
    RPW = ROWS // NW
    NCH = RPW // CH
    boff = batch * N
    mesh = plsc.VectorSubcoreMesh(core_axis_name="c", subcore_axis_name="s")
    cp = pltpu.CompilerParams()
    if "needs_layout_passes" in pltpu.CompilerParams.__dataclass_fields__:
        cp = dataclasses.replace(cp, needs_layout_passes=False)

    @functools.partial(
        pl.kernel,
        compiler_params=cp,
        out_type=[
            jax.ShapeDtypeStruct((ROWS, C), jnp.int32),
            jax.ShapeDtypeStruct((ROWS, XPAD), jnp.float32),
        ],
        mesh=mesh,
        scratch_types=[
            pltpu.VMEM((N,), jnp.float32),
            pltpu.VMEM((N,), jnp.float32),
            pltpu.VMEM((N,), jnp.float32),
            pltpu.VMEM((CH,), jnp.int32),
            pltpu.VMEM((CH,), jnp.int32),
            pltpu.VMEM((CH, C), jnp.int32),
            pltpu.VMEM((CH, C), jnp.int32),
            pltpu.VMEM((CH, XPAD), jnp.float32),
            pltpu.VMEM((CH, XPAD), jnp.float32),
            pltpu.SemaphoreType.DMA,
            pltpu.SemaphoreType.DMA,
            pltpu.SemaphoreType.DMA,
            pltpu.SemaphoreType.DMA,
            pltpu.SemaphoreType.DMA,
            pltpu.SemaphoreType.DMA,
        ],
    )
    def gather_k(tab_hbm, idx_hbm, xyzT_hbm, out_hbm, nx_hbm,
                 cx_v, cy_v, cz_v, idx0, idx1, rows0, rows1, nx0, nx1,
                 is0, is1, gs0, gs1, ws0, ws1):
        idx_v = (idx0, idx1)
        rows_v = (rows0, rows1)
        nx_v = (nx0, nx1)
        isem = (is0, is1)
        gsem = (gs0, gs1)
        wsem = (ws0, ws1)
        wid = lax.axis_index("s") * 2 + lax.axis_index("c")
        base = wid * RPW

        pltpu.sync_copy(xyzT_hbm.at[pl.ds((batch * 3 + 0) * N, N)], cx_v)
        pltpu.sync_copy(xyzT_hbm.at[pl.ds((batch * 3 + 1) * N, N)], cy_v)
        pltpu.sync_copy(xyzT_hbm.at[pl.ds((batch * 3 + 2) * N, N)], cz_v)

        zeros = jnp.zeros((L,), jnp.float32)

        def idx_src(i):
            return idx_hbm.at[pl.ds(base + i * CH, CH)]

        def rows_dst(i):
            return out_hbm.at[pl.ds(base + i * CH, CH)]

        def nx_dst(i):
            return nx_hbm.at[pl.ds(base + i * CH, CH)]

        def compute_nx(s):
            for j in range(CH // L):
                nb = idx_v[s][pl.ds(j * L, L)] - boff
                rows16 = lax.iota(jnp.int32, L) + (j * L)
                gx = plsc.load_gather(cx_v, [nb])
                gy = plsc.load_gather(cy_v, [nb])
                gz = plsc.load_gather(cz_v, [nb])
                plsc.store_scatter(nx_v[s], [rows16, jnp.full((L,), 0, jnp.int32)], gx)
                plsc.store_scatter(nx_v[s], [rows16, jnp.full((L,), 1, jnp.int32)], gy)
                plsc.store_scatter(nx_v[s], [rows16, jnp.full((L,), 2, jnp.int32)], gz)
                plsc.store_scatter(nx_v[s], [rows16, jnp.full((L,), 3, jnp.int32)], zeros)

        pltpu.async_copy(idx_src(0), idx_v[0], isem[0])

        @pl.loop(0, NCH, step=2)
        def _(i0):
            for par in range(2):
                s, o = par, 1 - par
                i = i0 + par
                @pl.when(i >= 1)
                def _():
                    pltpu.make_async_copy(
                        tab_hbm.at[idx_v[o]], rows_v[o], gsem[o]).wait()
                    pltpu.async_copy(rows_v[o], rows_dst(i - 1), wsem[o])
                    pltpu.async_copy(nx_v[o], nx_dst(i - 1), wsem[o])
                @pl.when(i + 1 < NCH)
                def _():
                    pltpu.async_copy(idx_src(i + 1), idx_v[o], isem[o])
                @pl.when(i >= 2)
                def _():
                    pltpu.make_async_copy(rows_v[s], rows_dst(i - 2), wsem[s]).wait()
                    pltpu.make_async_copy(nx_v[s], nx_dst(i - 2), wsem[s]).wait()
                pltpu.make_async_copy(idx_src(i), idx_v[s], isem[s]).wait()
                compute_nx(s)
                pltpu.async_copy(tab_hbm.at[idx_v[s]], rows_v[s], gsem[s])

        last = NCH - 1
        sl = last % 2
        pltpu.make_async_copy(tab_hbm.at[idx_v[sl]], rows_v[sl], gsem[sl]).wait()
        pltpu.async_copy(rows_v[sl], rows_dst(last), wsem[sl])
        pltpu.async_copy(nx_v[sl], nx_dst(last), wsem[sl])
        pltpu.make_async_copy(rows_v[1 - sl], rows_dst(last - 1), wsem[1 - sl]).wait()
        pltpu.make_async_copy(nx_v[1 - sl], nx_dst(last - 1), wsem[1 - sl]).wait()
        pltpu.make_async_copy(rows_v[sl], rows_dst(last), wsem[sl]).wait()
        pltpu.make_async_copy(nx_v[sl], nx_dst(last), wsem[sl]).wait()

    return gather_k(table, idx_flat, xyzT)


def _attention(x, xyzp3, gath3, nx3, Wq, bq, Wm1p, bm1, Wm2r, bm2r, Wo, bo,
               hd, BP):
    N, C = x.shape
    K = gath3.shape[0] // N
    scale = 1.0 / math.sqrt(hd)

    def body(x_ref, xyz_ref, g_ref, nx_ref, wq_ref, bq_ref, wm1_ref, bm1_ref,
             wm2_ref, bm2_ref, wo_ref, bo_ref, o_ref):
        bf = jnp.bfloat16
        xb = x_ref[...]
        q = jnp.dot(xb, wq_ref[...], preferred_element_type=jnp.float32) + bq_ref[...]
        q_bf = q.astype(bf)
        u = lax.bitcast_convert_type(g_ref[...], jnp.uint32)
        kn_bf = lax.bitcast_convert_type(
            lax.convert_element_type(u & jnp.uint32(0xFFFF), jnp.uint16), bf)
        vn_bf = lax.bitcast_convert_type(
            lax.convert_element_type(u >> 16, jnp.uint16), bf)

        nx = nx_ref[...]
        rel = xyz_ref[...][:, None, :] - nx.reshape(BP, K, XPAD)
        rel_bf = rel.reshape(BP * K, XPAD).astype(bf)
        h1 = jnp.dot(rel_bf, wm1_ref[...].astype(bf),
                     preferred_element_type=jnp.float32) + bm1_ref[...]
        h1_bf = jnp.maximum(h1, 0.0).astype(bf)
        hb = jnp.dot(h1_bf, wm2_ref[...].astype(bf),
                     preferred_element_type=jnp.float32) + bm2_ref[...]

        ce = lax.broadcasted_iota(jnp.int32, (C, C), 0) // hd
        je = lax.broadcasted_iota(jnp.int32, (C, C), 1) // hd
        E2 = jnp.where(ce == je, scale, 0.0).astype(bf)
        prod = (kn_bf.reshape(BP, K, C) * q_bf[:, None, :]).reshape(BP * K, C)
        s = jnp.dot(prod, E2, preferred_element_type=jnp.float32) + hb

        s3 = s.reshape(BP, K, C)
        m = jnp.max(s3, axis=1, keepdims=True)
        e = jnp.exp(s3 - m)
        den = jnp.sum(e, axis=1, keepdims=True)
        attn_bf = (e / den).astype(bf)

        oa = (attn_bf * vn_bf.reshape(BP, K, C)).sum(axis=1).astype(jnp.float32)
        o_ref[...] = jnp.dot(oa, wo_ref[...], preferred_element_type=jnp.float32) + bo_ref[...]

    grid = (N // BP,)
    return pl.pallas_call(
        body,
        grid=grid,
        in_specs=[
            pl.BlockSpec((BP, C), lambda i: (i, 0)),
            pl.BlockSpec((BP, XPAD), lambda i: (i, 0)),
            pl.BlockSpec((BP * K, C), lambda i: (i, 0)),
            pl.BlockSpec((BP * K, XPAD), lambda i: (i, 0)),
            pl.BlockSpec((C, C), lambda i: (0, 0)),
            pl.BlockSpec((1, C), lambda i: (0, 0)),
            pl.BlockSpec((XPAD, 32), lambda i: (0, 0)),
            pl.BlockSpec((1, 32), lambda i: (0, 0)),
            pl.BlockSpec((32, C), lambda i: (0, 0)),
            pl.BlockSpec((1, C), lambda i: (0, 0)),
            pl.BlockSpec((C, C), lambda i: (0, 0)),
            pl.BlockSpec((1, C), lambda i: (0, 0)),
        ],
        out_specs=pl.BlockSpec((BP, C), lambda i: (i, 0)),
        out_shape=jax.ShapeDtypeStruct((N, C), jnp.float32),
    )(x, xyzp3, gath3, nx3, Wq, bq, Wm1p, bm1, Wm2r, bm2r, Wo, bo)


def kernel(x, xyz, idx, Wq, bq, Wk, bk, Wv, bv, Wo, bo, Wm1, bm1, Wm2, bm2):
    B, N, C = x.shape
    K = idx.shape[2]
    H = Wm2.shape[1]

    x2 = x.reshape(B * N, C)
    idx2 = idx.reshape(B * N, K).astype(jnp.int32)
    xyzT = jnp.transpose(xyz, (0, 2, 1)).reshape(B * 3 * N)
    xyzp = jnp.pad(xyz, ((0, 0), (0, 0), (0, XPAD - 3)))
    Wm1p = jnp.pad(Wm1, ((0, XPAD - 3), (0, 0)))

    hd = C // H
    Wm2r = jnp.repeat(Wm2, hd, axis=1)
    bm2r = jnp.repeat(bm2.reshape(1, H), hd, axis=1)

    table, idxa = _build_table(x2, idx2, Wk, bk.reshape(1, C),
                               Wv, bv.reshape(1, C), N, BT=1024)

    idxa2 = idxa.reshape(B, N * K)
    outs = []
    for b in range(B):
        gath_b, nx_b = _sc_gather(table, idxa2[b], xyzT, N, K, b)
        out_b = _attention(x[b], xyzp[b], gath_b, nx_b,
                           Wq, bq.reshape(1, C), Wm1p, bm1.reshape(1, 32),
                           Wm2r, bm2r, Wo, bo.reshape(1, C), hd=hd, BP=256)
        outs.append(out_b)
    return jnp.stack(outs, axis=0)


# speedup vs baseline: 1.3848x; 1.0236x over previous
import dataclasses
import functools
import math

import jax
import jax.numpy as jnp
from jax import lax
from jax.experimental import pallas as pl
from jax.experimental.pallas import tpu as pltpu
from jax.experimental.pallas import tpu_sc as plsc

XPAD = 4


def _pack_bf16_pair(lo_f32, hi_f32):
    lo = lax.convert_element_type(
        lax.bitcast_convert_type(lo_f32.astype(jnp.bfloat16), jnp.uint16), jnp.uint32)
    hi = lax.convert_element_type(
        lax.bitcast_convert_type(hi_f32.astype(jnp.bfloat16), jnp.uint16), jnp.uint32)
    return lax.bitcast_convert_type((hi << 16) | lo, jnp.int32)


def _unpack_bf16_pair(packed_i32):
    u = lax.bitcast_convert_type(packed_i32, jnp.uint32)
    lo = lax.bitcast_convert_type(
        lax.convert_element_type(u & jnp.uint32(0xFFFF), jnp.uint16), jnp.bfloat16)
    hi = lax.bitcast_convert_type(
        lax.convert_element_type(u >> 16, jnp.uint16), jnp.bfloat16)
    return lo.astype(jnp.float32), hi.astype(jnp.float32)


def _build_table(x2, Wk, bk, Wv, bv, BT):
    NT, C = x2.shape

    def body(x_ref, wk_ref, bk_ref, wv_ref, bv_ref, tab_ref):
        xb = x_ref[...].astype(jnp.bfloat16)
        kf = jnp.dot(xb, wk_ref[...].astype(jnp.bfloat16),
                     preferred_element_type=jnp.float32) + bk_ref[...]
        vf = jnp.dot(xb, wv_ref[...].astype(jnp.bfloat16),
                     preferred_element_type=jnp.float32) + bv_ref[...]
        tab_ref[...] = _pack_bf16_pair(kf, vf)

    grid = (NT // BT,)
    return pl.pallas_call(
        body,
        grid=grid,
        in_specs=[
            pl.BlockSpec((BT, C), lambda i: (i, 0)),
            pl.BlockSpec((C, C), lambda i: (0, 0)),
            pl.BlockSpec((1, C), lambda i: (0, 0)),
            pl.BlockSpec((C, C), lambda i: (0, 0)),
            pl.BlockSpec((1, C), lambda i: (0, 0)),
        ],
        out_specs=pl.BlockSpec((BT, C), lambda i: (i, 0)),
        out_shape=jax.ShapeDtypeStruct((NT, C), jnp.int32),
    )(x2, Wk, bk, Wv, bv)


def _sc_gather(table, idx_flat, xyzT, N, K, batch):
    ROWS = idx_flat.shape[0]
    C = table.shape[1]
    NW = 32
    CH = 128
    L = 16
    RPW = ROWS // NW
    NCH = RPW // CH
    mesh = plsc.VectorSubcoreMesh(core_axis_name="c", subcore_axis_name="s")
    cp = pltpu.CompilerParams()
    if "needs_layout_passes" in pltpu.CompilerParams.__dataclass_fields__:
        cp = dataclasses.replace(cp, needs_layout_passes=False)

    @functools.partial(
        pl.kernel,
        compiler_params=cp,
        out_type=[
            jax.ShapeDtypeStruct((ROWS, C), jnp.int32),
            jax.ShapeDtypeStruct((ROWS, XPAD), jnp.float32),
        ],
        mesh=mesh,
        scratch_types=[
            pltpu.VMEM((N,), jnp.float32),
            pltpu.VMEM((N,), jnp.float32),
            pltpu.VMEM((N,), jnp.float32),
            pltpu.VMEM((CH,), jnp.int32),
            pltpu.VMEM((CH,), jnp.int32),
            pltpu.VMEM((CH, C), jnp.int32),
            pltpu.VMEM((CH, C), jnp.int32),
            pltpu.VMEM((CH, XPAD), jnp.float32),
            pltpu.VMEM((CH, XPAD), jnp.float32),
            pltpu.SemaphoreType.DMA,
            pltpu.SemaphoreType.DMA,
            pltpu.SemaphoreType.DMA,
            pltpu.SemaphoreType.DMA,
            pltpu.SemaphoreType.DMA,
            pltpu.SemaphoreType.DMA,
        ],
    )
    def gather_k(tab_hbm, idx_hbm, xyzT_hbm, out_hbm, nx_hbm,
                 cx_v, cy_v, cz_v, idx0, idx1, rows0, rows1, nx0, nx1,
                 is0, is1, gs0, gs1, ws0, ws1):
        idx_v = (idx0, idx1)
        rows_v = (rows0, rows1)
        nx_v = (nx0, nx1)
        isem = (is0, is1)
        gsem = (gs0, gs1)
        wsem = (ws0, ws1)
        wid = lax.axis_index("s") * 2 + lax.axis_index("c")
        base = wid * RPW

        pltpu.sync_copy(xyzT_hbm.at[pl.ds((batch * 3 + 0) * N, N)], cx_v)
        pltpu.sync_copy(xyzT_hbm.at[pl.ds((batch * 3 + 1) * N, N)], cy_v)
        pltpu.sync_copy(xyzT_hbm.at[pl.ds((batch * 3 + 2) * N, N)], cz_v)

        zeros = jnp.zeros((L,), jnp.float32)

        def idx_src(i):
            return idx_hbm.at[pl.ds(base + i * CH, CH)]

        def rows_dst(i):
            return out_hbm.at[pl.ds(base + i * CH, CH)]

        def nx_dst(i):
            return nx_hbm.at[pl.ds(base + i * CH, CH)]

        def compute_nx(s):
            for j in range(CH // L):
                nb = idx_v[s][pl.ds(j * L, L)]
                rows16 = lax.iota(jnp.int32, L) + (j * L)
                gx = plsc.load_gather(cx_v, [nb])
                gy = plsc.load_gather(cy_v, [nb])
                gz = plsc.load_gather(cz_v, [nb])
                plsc.store_scatter(nx_v[s], [rows16, jnp.full((L,), 0, jnp.int32)], gx)
                plsc.store_scatter(nx_v[s], [rows16, jnp.full((L,), 1, jnp.int32)], gy)
                plsc.store_scatter(nx_v[s], [rows16, jnp.full((L,), 2, jnp.int32)], gz)
                plsc.store_scatter(nx_v[s], [rows16, jnp.full((L,), 3, jnp.int32)], zeros)

        pltpu.async_copy(idx_src(0), idx_v[0], isem[0])

        @pl.loop(0, NCH, step=2)
        def _(i0):
            for par in range(2):
                s, o = par, 1 - par
                i = i0 + par
                @pl.when(i >= 1)
                def _():
                    pltpu.make_async_copy(
                        tab_hbm.at[idx_v[o]], rows_v[o], gsem[o]).wait()
                    pltpu.async_copy(rows_v[o], rows_dst(i - 1), wsem[o])
                    pltpu.async_copy(nx_v[o], nx_dst(i - 1), wsem[o])
                @pl.when(i + 1 < NCH)
                def _():
                    pltpu.async_copy(idx_src(i + 1), idx_v[o], isem[o])
                @pl.when(i >= 2)
                def _():
                    pltpu.make_async_copy(rows_v[s], rows_dst(i - 2), wsem[s]).wait()
                    pltpu.make_async_copy(nx_v[s], nx_dst(i - 2), wsem[s]).wait()
                pltpu.make_async_copy(idx_src(i), idx_v[s], isem[s]).wait()
                compute_nx(s)
                pltpu.async_copy(tab_hbm.at[idx_v[s]], rows_v[s], gsem[s])

        last = NCH - 1
        sl = last % 2
        pltpu.make_async_copy(tab_hbm.at[idx_v[sl]], rows_v[sl], gsem[sl]).wait()
        pltpu.async_copy(rows_v[sl], rows_dst(last), wsem[sl])
        pltpu.async_copy(nx_v[sl], nx_dst(last), wsem[sl])
        pltpu.make_async_copy(rows_v[1 - sl], rows_dst(last - 1), wsem[1 - sl]).wait()
        pltpu.make_async_copy(nx_v[1 - sl], nx_dst(last - 1), wsem[1 - sl]).wait()
        pltpu.make_async_copy(rows_v[sl], rows_dst(last), wsem[sl]).wait()
        pltpu.make_async_copy(nx_v[sl], nx_dst(last), wsem[sl]).wait()

    return gather_k(table, idx_flat, xyzT)


def _attention(x, xyzp3, gath3, nx3, Wq, bq, Wm1p, bm1, Wm2r, bm2r, Wo, bo,
               hd, BP):
    N, C = x.shape
    K = gath3.shape[0] // N
    scale = 1.0 / math.sqrt(hd)

    def body(x_ref, xyz_ref, g_ref, nx_ref, wq_ref, bq_ref, wm1_ref, bm1_ref,
             wm2_ref, bm2_ref, wo_ref, bo_ref, o_ref):
        bf = jnp.bfloat16
        xb = x_ref[...]
        q = jnp.dot(xb, wq_ref[...], preferred_element_type=jnp.float32) + bq_ref[...]
        q_bf = q.astype(bf)
        u = lax.bitcast_convert_type(g_ref[...], jnp.uint32)
        kn_bf = lax.bitcast_convert_type(
            lax.convert_element_type(u & jnp.uint32(0xFFFF), jnp.uint16), bf)
        vn_bf = lax.bitcast_convert_type(
            lax.convert_element_type(u >> 16, jnp.uint16), bf)

        nx = nx_ref[...]
        rel = xyz_ref[...][:, None, :] - nx.reshape(BP, K, XPAD)
        rel_bf = rel.reshape(BP * K, XPAD).astype(bf)
        h1 = jnp.dot(rel_bf, wm1_ref[...].astype(bf),
                     preferred_element_type=jnp.float32) + bm1_ref[...]
        h1_bf = jnp.maximum(h1, 0.0).astype(bf)
        hb = jnp.dot(h1_bf, wm2_ref[...].astype(bf),
                     preferred_element_type=jnp.float32) + bm2_ref[...]

        ce = lax.broadcasted_iota(jnp.int32, (C, C), 0) // hd
        je = lax.broadcasted_iota(jnp.int32, (C, C), 1) // hd
        E2 = jnp.where(ce == je, scale, 0.0).astype(bf)
        prod = (kn_bf.reshape(BP, K, C) * q_bf[:, None, :]).reshape(BP * K, C)
        s = jnp.dot(prod, E2, preferred_element_type=jnp.float32) + hb

        s3 = s.reshape(BP, K, C)
        m = jnp.max(s3, axis=1, keepdims=True)
        e = jnp.exp(s3 - m)
        den = jnp.sum(e, axis=1, keepdims=True)
        attn_bf = (e / den).astype(bf)

        oa = (attn_bf * vn_bf.reshape(BP, K, C)).sum(axis=1).astype(jnp.float32)
        o_ref[...] = jnp.dot(oa, wo_ref[...], preferred_element_type=jnp.float32) + bo_ref[...]

    grid = (N // BP,)
    return pl.pallas_call(
        body,
        grid=grid,
        in_specs=[
            pl.BlockSpec((BP, C), lambda i: (i, 0)),
            pl.BlockSpec((BP, XPAD), lambda i: (i, 0)),
            pl.BlockSpec((BP * K, C), lambda i: (i, 0)),
            pl.BlockSpec((BP * K, XPAD), lambda i: (i, 0)),
            pl.BlockSpec((C, C), lambda i: (0, 0)),
            pl.BlockSpec((1, C), lambda i: (0, 0)),
            pl.BlockSpec((XPAD, 32), lambda i: (0, 0)),
            pl.BlockSpec((1, 32), lambda i: (0, 0)),
            pl.BlockSpec((32, C), lambda i: (0, 0)),
            pl.BlockSpec((1, C), lambda i: (0, 0)),
            pl.BlockSpec((C, C), lambda i: (0, 0)),
            pl.BlockSpec((1, C), lambda i: (0, 0)),
        ],
        out_specs=pl.BlockSpec((BP, C), lambda i: (i, 0)),
        out_shape=jax.ShapeDtypeStruct((N, C), jnp.float32),
    )(x, xyzp3, gath3, nx3, Wq, bq, Wm1p, bm1, Wm2r, bm2r, Wo, bo)


def kernel(x, xyz, idx, Wq, bq, Wk, bk, Wv, bv, Wo, bo, Wm1, bm1, Wm2, bm2):
    B, N, C = x.shape
    K = idx.shape[2]
    H = Wm2.shape[1]

    xyzT = jnp.transpose(xyz, (0, 2, 1)).reshape(B * 3 * N)
    xyzp = jnp.pad(xyz, ((0, 0), (0, 0), (0, XPAD - 3)))
    Wm1p = jnp.pad(Wm1, ((0, XPAD - 3), (0, 0)))

    hd = C // H
    Wm2r = jnp.repeat(Wm2, hd, axis=1)
    bm2r = jnp.repeat(bm2.reshape(1, H), hd, axis=1)

    idx3 = idx.reshape(B, N * K).astype(jnp.int32)
    outs = []
    for b in range(B):
        table_b = _build_table(x[b], Wk, bk.reshape(1, C),
                               Wv, bv.reshape(1, C), BT=1024)
        gath_b, nx_b = _sc_gather(table_b, idx3[b], xyzT, N, K, b)
        out_b = _attention(x[b], xyzp[b], gath_b, nx_b,
                           Wq, bq.reshape(1, C), Wm1p, bm1.reshape(1, 32),
                           Wm2r, bm2r, Wo, bo.reshape(1, C), hd=hd, BP=256)
        outs.append(out_b)
    return jnp.stack(outs, axis=0)
